# baseline (speedup 1.0000x reference)
import jax
import jax.numpy as jnp
from jax.experimental import pallas as pl


def _hgnn_kernel(adj_ref, emb_ref, out_ref):
    h = pl.program_id(0)
    strip = adj_ref[...]
    emb = emb_ref[...]
    lat_blk = jax.lax.dot_general(
        strip, emb, (((0,), (0,)), ((), ())),
        preferred_element_type=jnp.float32)
    part = jax.lax.dot_general(
        strip, lat_blk, (((1,), (0,)), ((), ())),
        preferred_element_type=jnp.float32)

    @pl.when(h == 0)
    def _init():
        out_ref[...] = part

    @pl.when(h != 0)
    def _acc():
        out_ref[...] += part


def kernel(adj, embeds):
    n, hh = adj.shape
    d = embeds.shape[1]
    bh = 256
    return pl.pallas_call(
        _hgnn_kernel,
        grid=(hh // bh,),
        in_specs=[
            pl.BlockSpec((n, bh), lambda h: (0, h)),
            pl.BlockSpec((n, d), lambda h: (0, 0)),
        ],
        out_specs=pl.BlockSpec((n, d), lambda h: (0, 0)),
        out_shape=jax.ShapeDtypeStruct((n, d), jnp.float32),
    )(adj, embeds)


# speedup vs baseline: 1.4921x; 1.4921x over previous
import jax
import jax.numpy as jnp
from jax.experimental import pallas as pl


def _hgnn_kernel(adj_ref, emb_ref, out_ref):
    h = pl.program_id(0)
    strip = adj_ref[...].astype(jnp.bfloat16)
    emb = emb_ref[...].astype(jnp.bfloat16)
    lat_blk = jax.lax.dot_general(
        strip, emb, (((0,), (0,)), ((), ())),
        preferred_element_type=jnp.float32)
    part = jax.lax.dot_general(
        strip, lat_blk.astype(jnp.bfloat16), (((1,), (0,)), ((), ())),
        preferred_element_type=jnp.float32)

    @pl.when(h == 0)
    def _init():
        out_ref[...] = part

    @pl.when(h != 0)
    def _acc():
        out_ref[...] += part


def kernel(adj, embeds):
    n, hh = adj.shape
    d = embeds.shape[1]
    bh = 256
    return pl.pallas_call(
        _hgnn_kernel,
        grid=(hh // bh,),
        in_specs=[
            pl.BlockSpec((n, bh), lambda h: (0, h)),
            pl.BlockSpec((n, d), lambda h: (0, 0)),
        ],
        out_specs=pl.BlockSpec((n, d), lambda h: (0, 0)),
        out_shape=jax.ShapeDtypeStruct((n, d), jnp.float32),
    )(adj, embeds)
